# Initial kernel scaffold; baseline (speedup 1.0000x reference)
import jax
import jax.numpy as jnp
from jax.experimental import pallas as pl


def kernel(x, gate_w, gate_b, Wg, bg, Wu, bu, Wd, bd):
    raise NotImplementedError("write your pallas kernel here")


# speedup vs baseline: 1.3654x; 1.3654x over previous
import functools

import jax
import jax.numpy as jnp
from jax import lax
from jax.experimental import pallas as pl
from jax.experimental.pallas import tpu as pltpu

D_MODEL = 2048
D_FF = 5632
NUM_EXPERTS = 8
TOP_K = 2

TM = 256
FB = 512
NF = D_FF // FB
P_CAP = TOP_K * 2048 + NUM_EXPERTS * TM
NT = P_CAP // TM


def _mlp_body(te_ref, x_ref, wg_ref, bg_ref, wu_ref, bu_ref, wd_ref, bd_ref,
              y_ref):
    f = pl.program_id(1)
    dn = (((1,), (1,)), ((), ()))
    x = x_ref[...]
    h1 = lax.dot_general(x, wg_ref[0], dn,
                         preferred_element_type=jnp.float32) + bg_ref[0]
    h1 = h1 * jax.nn.sigmoid(h1)
    h2 = lax.dot_general(x, wu_ref[0], dn,
                         preferred_element_type=jnp.float32) + bu_ref[0]
    h = h1 * h2
    y = lax.dot_general(h, wd_ref[0], dn, preferred_element_type=jnp.float32)

    @pl.when(f == 0)
    def _():
        y_ref[...] = y + bd_ref[0]

    @pl.when(f != 0)
    def _():
        y_ref[...] = y_ref[...] + y


def _grouped_mlp(tile_expert, xg, Wg, bg, Wu, bu, Wd, bd):
    grid_spec = pltpu.PrefetchScalarGridSpec(
        num_scalar_prefetch=1,
        grid=(NT, NF),
        in_specs=[
            pl.BlockSpec((TM, D_MODEL), lambda m, f, te: (m, 0)),
            pl.BlockSpec((1, FB, D_MODEL), lambda m, f, te: (te[m], f, 0)),
            pl.BlockSpec((1, 1, FB), lambda m, f, te: (te[m], 0, f)),
            pl.BlockSpec((1, FB, D_MODEL), lambda m, f, te: (te[m], f, 0)),
            pl.BlockSpec((1, 1, FB), lambda m, f, te: (te[m], 0, f)),
            pl.BlockSpec((1, D_MODEL, FB), lambda m, f, te: (te[m], 0, f)),
            pl.BlockSpec((1, 1, D_MODEL), lambda m, f, te: (te[m], 0, 0)),
        ],
        out_specs=pl.BlockSpec((TM, D_MODEL), lambda m, f, te: (m, 0)),
    )
    return pl.pallas_call(
        _mlp_body,
        grid_spec=grid_spec,
        out_shape=jax.ShapeDtypeStruct((P_CAP, D_MODEL), jnp.float32),
        compiler_params=pltpu.CompilerParams(
            dimension_semantics=("arbitrary", "arbitrary")),
    )(tile_expert, xg, Wg, bg, Wu, bu, Wd, bd)


def kernel(x, gate_w, gate_b, Wg, bg, Wu, bu, Wd, bd):
    b, s, d = x.shape
    T = b * s
    xf = x.reshape(T, d)

    logits = xf @ gate_w.T + gate_b
    probs = jax.nn.softmax(logits.astype(jnp.float32), axis=1)
    rw, sel = lax.top_k(probs, TOP_K)
    rw = rw / jnp.sum(rw, axis=-1, keepdims=True)

    flat_e = sel.reshape(-1).astype(jnp.int32)
    A = flat_e.shape[0]
    sort_idx = jnp.argsort(flat_e, stable=True)
    inv = jnp.zeros((A,), jnp.int32).at[sort_idx].set(
        jnp.arange(A, dtype=jnp.int32))
    counts = jnp.bincount(flat_e, length=NUM_EXPERTS).astype(jnp.int32)
    raw_off = jnp.concatenate(
        [jnp.zeros((1,), jnp.int32), jnp.cumsum(counts)[:-1]])
    pad_counts = ((counts + TM - 1) // TM) * TM
    pad_end = jnp.cumsum(pad_counts)
    pad_off = jnp.concatenate([jnp.zeros((1,), jnp.int32), pad_end[:-1]])
    pos = pad_off[flat_e] + (inv - raw_off[flat_e])
    tok = (jnp.arange(A, dtype=jnp.int32) // TOP_K)
    perm_tok = jnp.zeros((P_CAP,), jnp.int32).at[pos].set(tok)
    xg = xf[perm_tok]

    tile_start = jnp.arange(NT, dtype=jnp.int32) * TM
    te = jnp.searchsorted(pad_end, tile_start, side='right').astype(jnp.int32)
    te = jnp.minimum(te, NUM_EXPERTS - 1)

    y = _grouped_mlp(te, xg, Wg, bg.reshape(NUM_EXPERTS, 1, D_FF),
                     Wu, bu.reshape(NUM_EXPERTS, 1, D_FF),
                     Wd, bd.reshape(NUM_EXPERTS, 1, D_MODEL))

    p = pos.reshape(T, TOP_K)
    final = rw[:, 0:1] * y[p[:, 0]] + rw[:, 1:2] * y[p[:, 1]]
    return final.reshape(b, s, d), logits
